# Initial kernel scaffold; baseline (speedup 1.0000x reference)
import jax
import jax.numpy as jnp
from jax.experimental import pallas as pl


def kernel(x, edge_index, W1, b1, W2, b2, Wl1, bl1, Wl2, bl2):
    raise NotImplementedError("write your pallas kernel here")


# speedup vs baseline: 8.4258x; 8.4258x over previous
import functools

import jax
import jax.numpy as jnp
from jax import lax
from jax.experimental import pallas as pl
from jax.experimental.pallas import tpu as pltpu
from jax.experimental.pallas import tpu_sc as plsc

N_NODES = 10000
D = 128
N_EDGES = 320000

NC = 2
NS = 16
NW = NC * NS

CHUNK = 128
E_TILE = 10240
NCH = E_TILE // CHUNK
NE_PAD = NW * E_TILE
NPAD = 10240
ZROWS = NPAD // NS


def _sc_degree_impl(dst_hbm, out_hbm, idx_v, ones_v, deg_sp, sem):
    cid = lax.axis_index("c")
    sid = lax.axis_index("s")
    wid = cid * NS + sid

    zeros16 = jnp.zeros((16,), jnp.float32)
    ones16 = jnp.ones((16,), jnp.float32)

    def fillz(i, _):
        for j in range(D // 16):
            ones_v[i, pl.ds(j * 16, 16)] = zeros16
        return 0

    lax.fori_loop(0, CHUNK, fillz, 0)

    for k in range(ZROWS // CHUNK):
        pltpu.sync_copy(ones_v, deg_sp.at[pl.ds(sid * ZROWS + k * CHUNK, CHUNK)])
    plsc.subcore_barrier()

    def fill1(i, _):
        for j in range(D // 16):
            ones_v[i, pl.ds(j * 16, 16)] = ones16
        return 0

    lax.fori_loop(0, CHUNK, fill1, 0)

    pltpu.sync_copy(dst_hbm.at[wid], idx_v)

    def body(c, _):
        pltpu.sync_copy(ones_v, deg_sp.at[idx_v.at[c]], add=True)
        return 0

    lax.fori_loop(0, NCH, body, 0)
    plsc.subcore_barrier()

    pltpu.sync_copy(
        deg_sp.at[pl.ds(sid * ZROWS, ZROWS)],
        out_hbm.at[cid, pl.ds(sid * ZROWS, ZROWS)],
    )


HALF = NCH // 2


def _sc_spmm_impl(g_hbm, src_hbm, dst_hbm, out_hbm,
                  src_v, dst_v, rows0, rows1, acc_sp, sem0, sem1):
    cid = lax.axis_index("c")
    sid = lax.axis_index("s")
    wid = cid * NS + sid

    zeros16 = jnp.zeros((16,), jnp.float32)

    def fill(i, _):
        for j in range(D // 16):
            rows0[i, pl.ds(j * 16, 16)] = zeros16
        return 0

    lax.fori_loop(0, CHUNK, fill, 0)

    for k in range(ZROWS // CHUNK):
        pltpu.sync_copy(rows0, acc_sp.at[pl.ds(sid * ZROWS + k * CHUNK, CHUNK)])
    plsc.subcore_barrier()

    rows = (rows0, rows1)
    sems = (sem0, sem1)

    for h in range(2):
        pltpu.sync_copy(src_hbm.at[wid, pl.ds(h * HALF, HALF)], src_v)
        pltpu.sync_copy(dst_hbm.at[wid, pl.ds(h * HALF, HALF)], dst_v)

        pltpu.async_copy(g_hbm.at[src_v.at[0]], rows[0], sems[0])

        def body(cc, _):
            c0 = cc * 2
            pltpu.async_copy(g_hbm.at[src_v.at[c0 + 1]], rows[1], sems[1])
            pltpu.make_async_copy(g_hbm.at[src_v.at[c0]], rows[0], sems[0]).wait()
            pltpu.sync_copy(rows[0], acc_sp.at[dst_v.at[c0]], add=True)

            @pl.when(c0 + 2 < HALF)
            def _():
                pltpu.async_copy(g_hbm.at[src_v.at[c0 + 2]], rows[0], sems[0])

            pltpu.make_async_copy(g_hbm.at[src_v.at[c0 + 1]], rows[1], sems[1]).wait()
            pltpu.sync_copy(rows[1], acc_sp.at[dst_v.at[c0 + 1]], add=True)
            return 0

        lax.fori_loop(0, HALF // 2, body, 0)
    plsc.subcore_barrier()

    pltpu.sync_copy(
        acc_sp.at[pl.ds(sid * ZROWS, ZROWS)],
        out_hbm.at[cid, pl.ds(sid * ZROWS, ZROWS)],
    )


@functools.cache
def _sc_kernels():
    mesh = plsc.VectorSubcoreMesh(
        core_axis_name="c", subcore_axis_name="s", num_cores=NC, num_subcores=NS
    )
    sc_degree = pl.kernel(
        _sc_degree_impl,
        out_type=jax.ShapeDtypeStruct((NC, NPAD, D), jnp.float32),
        mesh=mesh,
        scratch_types=[
            pltpu.VMEM((NCH, CHUNK), jnp.int32),
            pltpu.VMEM((CHUNK, D), jnp.float32),
            pltpu.VMEM_SHARED((NPAD, D), jnp.float32),
            pltpu.SemaphoreType.DMA,
        ],
    )
    sc_spmm = pl.kernel(
        _sc_spmm_impl,
        out_type=jax.ShapeDtypeStruct((NC, NPAD, D), jnp.float32),
        mesh=mesh,
        scratch_types=[
            pltpu.VMEM((HALF, CHUNK), jnp.int32),
            pltpu.VMEM((HALF, CHUNK), jnp.int32),
            pltpu.VMEM((CHUNK, D), jnp.float32),
            pltpu.VMEM((CHUNK, D), jnp.float32),
            pltpu.VMEM_SHARED((NPAD, D), jnp.float32),
            pltpu.SemaphoreType.DMA,
            pltpu.SemaphoreType.DMA,
        ],
    )
    return sc_degree, sc_spmm


BLK = 1024
GRID = NPAD // BLK


def _tc_scale_body(degp_ref, x_ref, w_ref, g_ref, dinv_ref):
    deg = degp_ref[0, :, 0:1] + degp_ref[1, :, 0:1] + 1.0
    dinv = lax.rsqrt(deg)
    dinv_b = jnp.broadcast_to(dinv, (BLK, D))
    dinv_ref[...] = dinv_b
    h = jnp.dot(x_ref[...], w_ref[...], preferred_element_type=jnp.float32)
    g_ref[...] = h * dinv_b


def _tc_scale(degp, xp, W):
    return pl.pallas_call(
        _tc_scale_body,
        grid=(GRID,),
        in_specs=[
            pl.BlockSpec((NC, BLK, D), lambda i: (0, i, 0)),
            pl.BlockSpec((BLK, D), lambda i: (i, 0)),
            pl.BlockSpec((D, D), lambda i: (0, 0)),
        ],
        out_specs=[
            pl.BlockSpec((BLK, D), lambda i: (i, 0)),
            pl.BlockSpec((BLK, D), lambda i: (i, 0)),
        ],
        out_shape=[
            jax.ShapeDtypeStruct((NPAD, D), jnp.float32),
            jax.ShapeDtypeStruct((NPAD, D), jnp.float32),
        ],
    )(degp, xp, W)


def _tc_layer_body(acc_ref, g_ref, dinv_ref, b_ref, w_ref, out_ref):
    dinv = dinv_ref[...]
    h = dinv * (acc_ref[0] + acc_ref[1] + g_ref[...]) + b_ref[...]
    h = jnp.maximum(h, 0.0)
    out_ref[...] = jnp.dot(h, w_ref[...], preferred_element_type=jnp.float32) * dinv


def _tc_layer(accp, g, dinv_b, bvec, W):
    return pl.pallas_call(
        _tc_layer_body,
        grid=(GRID,),
        in_specs=[
            pl.BlockSpec((NC, BLK, D), lambda i: (0, i, 0)),
            pl.BlockSpec((BLK, D), lambda i: (i, 0)),
            pl.BlockSpec((BLK, D), lambda i: (i, 0)),
            pl.BlockSpec((1, D), lambda i: (0, 0)),
            pl.BlockSpec((D, D), lambda i: (0, 0)),
        ],
        out_specs=pl.BlockSpec((BLK, D), lambda i: (i, 0)),
        out_shape=jax.ShapeDtypeStruct((NPAD, D), jnp.float32),
    )(accp, g, dinv_b, bvec, W)


def _tc_head_body(acc_ref, g_ref, dinv_ref, b2_ref, wl1_ref, bl1_ref,
                  wl2_ref, bl2_ref, out_ref):
    dinv = dinv_ref[...]
    h = dinv * (acc_ref[0] + acc_ref[1] + g_ref[...]) + b2_ref[...]
    h = jnp.maximum(h, 0.0)
    h = jnp.dot(h, wl1_ref[...], preferred_element_type=jnp.float32) + bl1_ref[...]
    h = jnp.maximum(h, 0.0)
    out_ref[...] = (
        jnp.dot(h, wl2_ref[...], preferred_element_type=jnp.float32) + bl2_ref[...]
    )


def _tc_head(accp, g, dinv_b, b2vec, Wl1p, bl1p, Wl2p, bl2p):
    return pl.pallas_call(
        _tc_head_body,
        grid=(GRID,),
        in_specs=[
            pl.BlockSpec((NC, BLK, D), lambda i: (0, i, 0)),
            pl.BlockSpec((BLK, D), lambda i: (i, 0)),
            pl.BlockSpec((BLK, D), lambda i: (i, 0)),
            pl.BlockSpec((1, D), lambda i: (0, 0)),
            pl.BlockSpec((D, D), lambda i: (0, 0)),
            pl.BlockSpec((1, D), lambda i: (0, 0)),
            pl.BlockSpec((D, D), lambda i: (0, 0)),
            pl.BlockSpec((1, D), lambda i: (0, 0)),
        ],
        out_specs=pl.BlockSpec((BLK, D), lambda i: (i, 0)),
        out_shape=jax.ShapeDtypeStruct((NPAD, D), jnp.float32),
    )(accp, g, dinv_b, b2vec, Wl1p, bl1p, Wl2p, bl2p)


@jax.jit
def kernel(x, edge_index, W1, b1, W2, b2, Wl1, bl1, Wl2, bl2):
    src = edge_index[0].astype(jnp.int32)
    dst = edge_index[1].astype(jnp.int32)
    npad_e = NE_PAD - N_EDGES
    pad_src = jnp.full((npad_e,), N_NODES, jnp.int32)
    pad_dst = N_NODES + (jnp.arange(npad_e, dtype=jnp.int32) % (NPAD - N_NODES))
    src3 = jnp.concatenate([src, pad_src]).reshape(NW, NCH, CHUNK)
    dst3 = jnp.concatenate([dst, pad_dst]).reshape(NW, NCH, CHUNK)

    xp = jnp.zeros((NPAD, D), jnp.float32).at[:N_NODES].set(x)
    b1v = b1.reshape(1, D)
    b2v = b2.reshape(1, D)
    Wl1p = jnp.zeros((D, D), jnp.float32).at[:, :64].set(Wl1)
    bl1p = jnp.zeros((1, D), jnp.float32).at[0, :64].set(bl1)
    Wl2p = jnp.zeros((D, D), jnp.float32).at[:64, 0:1].set(Wl2)
    bl2p = jnp.zeros((1, D), jnp.float32).at[0, 0:1].set(bl2)

    sc_degree, sc_spmm = _sc_kernels()
    degp = sc_degree(dst3)
    g1, dinv_b = _tc_scale(degp, xp, W1)
    acc1 = sc_spmm(g1, src3, dst3)
    g2 = _tc_layer(acc1, g1, dinv_b, b1v, W2)
    acc2 = sc_spmm(g2, src3, dst3)
    out = _tc_head(acc2, g2, dinv_b, b2v, Wl1p, bl1p, Wl2p, bl2p)
    return out[:N_NODES, 0:1]


# speedup vs baseline: 9.3226x; 1.1064x over previous
import functools

import jax
import jax.numpy as jnp
from jax import lax
from jax.experimental import pallas as pl
from jax.experimental.pallas import tpu as pltpu
from jax.experimental.pallas import tpu_sc as plsc

N_NODES = 10000
D = 128
N_EDGES = 320000

NC = 2
NS = 16
NW = NC * NS

CHUNK = 128
E_TILE = 10240
NCH = E_TILE // CHUNK
NE_PAD = NW * E_TILE
NPAD = 10240
ZROWS = NPAD // NS


def _sc_degree_impl(dst_hbm, out_hbm, idx_v, ones_v, deg_sp, sem):
    cid = lax.axis_index("c")
    sid = lax.axis_index("s")
    wid = cid * NS + sid

    zeros16 = jnp.zeros((16,), jnp.float32)
    ones16 = jnp.ones((16,), jnp.float32)

    def fillz(i, _):
        for j in range(D // 16):
            ones_v[i, pl.ds(j * 16, 16)] = zeros16
        return 0

    lax.fori_loop(0, CHUNK, fillz, 0)

    for k in range(ZROWS // CHUNK):
        pltpu.sync_copy(ones_v, deg_sp.at[pl.ds(sid * ZROWS + k * CHUNK, CHUNK)])
    plsc.subcore_barrier()

    def fill1(i, _):
        for j in range(D // 16):
            ones_v[i, pl.ds(j * 16, 16)] = ones16
        return 0

    lax.fori_loop(0, CHUNK, fill1, 0)

    pltpu.sync_copy(dst_hbm.at[wid], idx_v)

    def body(c, _):
        pltpu.sync_copy(ones_v, deg_sp.at[idx_v.at[c]], add=True)
        return 0

    lax.fori_loop(0, NCH, body, 0)
    plsc.subcore_barrier()

    pltpu.sync_copy(
        deg_sp.at[pl.ds(sid * ZROWS, ZROWS)],
        out_hbm.at[cid, pl.ds(sid * ZROWS, ZROWS)],
    )


ECHUNK = 64
ENCH = E_TILE // ECHUNK
EQ = ENCH // 4
NBUF = 4


def _sc_spmm_impl(g_hbm, src_hbm, dst_hbm, out_hbm,
                  src_v, dst_v, rows0, rows1, rows2, rows3,
                  acc_sp, gs0, gs1, gs2, gs3):
    cid = lax.axis_index("c")
    sid = lax.axis_index("s")
    wid = cid * NS + sid

    zeros16 = jnp.zeros((16,), jnp.float32)

    def fill(i, _):
        for j in range(D // 16):
            rows0[i, pl.ds(j * 16, 16)] = zeros16
        return 0

    lax.fori_loop(0, ECHUNK, fill, 0)

    for k in range(ZROWS // ECHUNK):
        pltpu.sync_copy(rows0, acc_sp.at[pl.ds(sid * ZROWS + k * ECHUNK, ECHUNK)])
    plsc.subcore_barrier()

    rows = (rows0, rows1, rows2, rows3)
    gsems = (gs0, gs1, gs2, gs3)

    for h in range(4):
        pltpu.sync_copy(src_hbm.at[wid, pl.ds(h * EQ, EQ)], src_v)
        pltpu.sync_copy(dst_hbm.at[wid, pl.ds(h * EQ, EQ)], dst_v)

        for k in range(NBUF):
            pltpu.async_copy(g_hbm.at[src_v.at[k]], rows[k], gsems[k])

        def body(cc, _):
            c0 = cc * NBUF
            for k in range(NBUF):
                pltpu.make_async_copy(
                    g_hbm.at[src_v.at[c0 + k]], rows[k], gsems[k]).wait()
                pltpu.sync_copy(rows[k], acc_sp.at[dst_v.at[c0 + k]], add=True)

                @pl.when(c0 + k + NBUF < EQ)
                def _():
                    pltpu.async_copy(
                        g_hbm.at[src_v.at[c0 + k + NBUF]], rows[k], gsems[k])
            return 0

        lax.fori_loop(0, EQ // NBUF, body, 0)
    plsc.subcore_barrier()

    pltpu.sync_copy(
        acc_sp.at[pl.ds(sid * ZROWS, ZROWS)],
        out_hbm.at[cid, pl.ds(sid * ZROWS, ZROWS)],
    )


@functools.cache
def _sc_kernels():
    mesh = plsc.VectorSubcoreMesh(
        core_axis_name="c", subcore_axis_name="s", num_cores=NC, num_subcores=NS
    )
    sc_degree = pl.kernel(
        _sc_degree_impl,
        out_type=jax.ShapeDtypeStruct((NC, NPAD, D), jnp.float32),
        mesh=mesh,
        scratch_types=[
            pltpu.VMEM((NCH, CHUNK), jnp.int32),
            pltpu.VMEM((CHUNK, D), jnp.float32),
            pltpu.VMEM_SHARED((NPAD, D), jnp.float32),
            pltpu.SemaphoreType.DMA,
        ],
    )
    sc_spmm = pl.kernel(
        _sc_spmm_impl,
        out_type=jax.ShapeDtypeStruct((NC, NPAD, D), jnp.float32),
        mesh=mesh,
        scratch_types=[
            pltpu.VMEM((EQ, ECHUNK), jnp.int32),
            pltpu.VMEM((EQ, ECHUNK), jnp.int32),
            pltpu.VMEM((ECHUNK, D), jnp.float32),
            pltpu.VMEM((ECHUNK, D), jnp.float32),
            pltpu.VMEM((ECHUNK, D), jnp.float32),
            pltpu.VMEM((ECHUNK, D), jnp.float32),
            pltpu.VMEM_SHARED((NPAD, D), jnp.float32),
            pltpu.SemaphoreType.DMA,
            pltpu.SemaphoreType.DMA,
            pltpu.SemaphoreType.DMA,
            pltpu.SemaphoreType.DMA,
        ],
    )
    return sc_degree, sc_spmm


BLK = 1024
GRID = NPAD // BLK


def _tc_scale_body(degp_ref, x_ref, w_ref, g_ref, dinv_ref):
    deg = degp_ref[0, :, 0:1] + degp_ref[1, :, 0:1] + 1.0
    dinv = lax.rsqrt(deg)
    dinv_b = jnp.broadcast_to(dinv, (BLK, D))
    dinv_ref[...] = dinv_b
    h = jnp.dot(x_ref[...], w_ref[...], preferred_element_type=jnp.float32)
    g_ref[...] = h * dinv_b


def _tc_scale(degp, xp, W):
    return pl.pallas_call(
        _tc_scale_body,
        grid=(GRID,),
        in_specs=[
            pl.BlockSpec((NC, BLK, D), lambda i: (0, i, 0)),
            pl.BlockSpec((BLK, D), lambda i: (i, 0)),
            pl.BlockSpec((D, D), lambda i: (0, 0)),
        ],
        out_specs=[
            pl.BlockSpec((BLK, D), lambda i: (i, 0)),
            pl.BlockSpec((BLK, D), lambda i: (i, 0)),
        ],
        out_shape=[
            jax.ShapeDtypeStruct((NPAD, D), jnp.float32),
            jax.ShapeDtypeStruct((NPAD, D), jnp.float32),
        ],
    )(degp, xp, W)


def _tc_layer_body(acc_ref, g_ref, dinv_ref, b_ref, w_ref, out_ref):
    dinv = dinv_ref[...]
    h = dinv * (acc_ref[0] + acc_ref[1] + g_ref[...]) + b_ref[...]
    h = jnp.maximum(h, 0.0)
    out_ref[...] = jnp.dot(h, w_ref[...], preferred_element_type=jnp.float32) * dinv


def _tc_layer(accp, g, dinv_b, bvec, W):
    return pl.pallas_call(
        _tc_layer_body,
        grid=(GRID,),
        in_specs=[
            pl.BlockSpec((NC, BLK, D), lambda i: (0, i, 0)),
            pl.BlockSpec((BLK, D), lambda i: (i, 0)),
            pl.BlockSpec((BLK, D), lambda i: (i, 0)),
            pl.BlockSpec((1, D), lambda i: (0, 0)),
            pl.BlockSpec((D, D), lambda i: (0, 0)),
        ],
        out_specs=pl.BlockSpec((BLK, D), lambda i: (i, 0)),
        out_shape=jax.ShapeDtypeStruct((NPAD, D), jnp.float32),
    )(accp, g, dinv_b, bvec, W)


def _tc_head_body(acc_ref, g_ref, dinv_ref, b2_ref, wl1_ref, bl1_ref,
                  wl2_ref, bl2_ref, out_ref):
    dinv = dinv_ref[...]
    h = dinv * (acc_ref[0] + acc_ref[1] + g_ref[...]) + b2_ref[...]
    h = jnp.maximum(h, 0.0)
    h = jnp.dot(h, wl1_ref[...], preferred_element_type=jnp.float32) + bl1_ref[...]
    h = jnp.maximum(h, 0.0)
    out_ref[...] = (
        jnp.dot(h, wl2_ref[...], preferred_element_type=jnp.float32) + bl2_ref[...]
    )


def _tc_head(accp, g, dinv_b, b2vec, Wl1p, bl1p, Wl2p, bl2p):
    return pl.pallas_call(
        _tc_head_body,
        grid=(GRID,),
        in_specs=[
            pl.BlockSpec((NC, BLK, D), lambda i: (0, i, 0)),
            pl.BlockSpec((BLK, D), lambda i: (i, 0)),
            pl.BlockSpec((BLK, D), lambda i: (i, 0)),
            pl.BlockSpec((1, D), lambda i: (0, 0)),
            pl.BlockSpec((D, D), lambda i: (0, 0)),
            pl.BlockSpec((1, D), lambda i: (0, 0)),
            pl.BlockSpec((D, D), lambda i: (0, 0)),
            pl.BlockSpec((1, D), lambda i: (0, 0)),
        ],
        out_specs=pl.BlockSpec((BLK, D), lambda i: (i, 0)),
        out_shape=jax.ShapeDtypeStruct((NPAD, D), jnp.float32),
    )(accp, g, dinv_b, b2vec, Wl1p, bl1p, Wl2p, bl2p)


@jax.jit
def kernel(x, edge_index, W1, b1, W2, b2, Wl1, bl1, Wl2, bl2):
    src = edge_index[0].astype(jnp.int32)
    dst = edge_index[1].astype(jnp.int32)
    npad_e = NE_PAD - N_EDGES
    pad_src = jnp.full((npad_e,), N_NODES, jnp.int32)
    pad_dst = N_NODES + (jnp.arange(npad_e, dtype=jnp.int32) % (NPAD - N_NODES))
    src_all = jnp.concatenate([src, pad_src])
    dst_all = jnp.concatenate([dst, pad_dst])
    dst3 = dst_all.reshape(NW, NCH, CHUNK)
    src3e = src_all.reshape(NW, ENCH, ECHUNK)
    dst3e = dst_all.reshape(NW, ENCH, ECHUNK)

    xp = jnp.zeros((NPAD, D), jnp.float32).at[:N_NODES].set(x)
    b1v = b1.reshape(1, D)
    b2v = b2.reshape(1, D)
    Wl1p = jnp.zeros((D, D), jnp.float32).at[:, :64].set(Wl1)
    bl1p = jnp.zeros((1, D), jnp.float32).at[0, :64].set(bl1)
    Wl2p = jnp.zeros((D, D), jnp.float32).at[:64, 0:1].set(Wl2)
    bl2p = jnp.zeros((1, D), jnp.float32).at[0, 0:1].set(bl2)

    sc_degree, sc_spmm = _sc_kernels()
    degp = sc_degree(dst3)
    g1, dinv_b = _tc_scale(degp, xp, W1)
    acc1 = sc_spmm(g1, src3e, dst3e)
    g2 = _tc_layer(acc1, g1, dinv_b, b1v, W2)
    acc2 = sc_spmm(g2, src3e, dst3e)
    out = _tc_head(acc2, g2, dinv_b, b2v, Wl1p, bl1p, Wl2p, bl2p)
    return out[:N_NODES, 0:1]
